# baseline (speedup 1.0000x reference)
import functools

import jax
import jax.numpy as jnp
from jax import lax
from jax.experimental import pallas as pl
from jax.experimental.pallas import tpu as pltpu
from jax.experimental.pallas import tpu_sc as plsc

D_REPR = 64
TOP_K = 3
THRESH = 0.95
MEM = 5000
T = 200
B = 4096
BT = 256
ROW_W = 12800

NC = 2
NS = 16
NW = NC * NS
RPW = B // NW

_NEG_INF = float("-inf")

_FULL_BATCHES = T // 16
_TAIL = T - _FULL_BATCHES * 16


def _row_stats(row_ref, scr_ref, xf_ref, lane_iota):
    s_acc = jnp.zeros((16,), jnp.float32)
    mx_acc = jnp.full((16,), _NEG_INF, jnp.float32)
    mn_acc = jnp.full((16,), float("inf"), jnp.float32)
    inv_d = jnp.float32(1.0 / 64.0)
    tail_mask = lane_iota < _TAIL

    n_batches = _FULL_BATCHES + 1
    for b in range(n_batches):
        nk = 16 if b < _FULL_BATCHES else _TAIL
        for k in range(nk):
            base = (b * 16 + k) * 64
            v = (row_ref[pl.ds(base, 16)] + row_ref[pl.ds(base + 16, 16)]
                 + row_ref[pl.ds(base + 32, 16)]
                 + row_ref[pl.ds(base + 48, 16)])
            scr_ref[k, :] = v
        csum = jnp.zeros((16,), jnp.float32)
        colj = jnp.zeros((16,), jnp.int32)
        for j in range(16):
            col = plsc.load_gather(
                scr_ref, [lane_iota, colj + jnp.int32(j)])
            csum = csum + col
        g = csum * inv_d
        if b < _FULL_BATCHES:
            s_acc = s_acc + g
            mx_acc = jnp.maximum(mx_acc, g)
            mn_acc = jnp.minimum(mn_acc, g)
            xf_ref[b, :] = g
        else:
            s_acc = s_acc + jnp.where(tail_mask, g, jnp.float32(0.0))
            mx_acc = jnp.maximum(mx_acc, jnp.where(tail_mask, g, _NEG_INF))
            mn_acc = jnp.minimum(mn_acc,
                                 jnp.where(tail_mask, g, float("inf")))
            xf_ref[b, :] = g

    mean = jnp.sum(s_acc) * jnp.float32(1.0 / T)
    bc_mu = jnp.full((16,), mean, jnp.float32)
    acc2 = jnp.zeros((16,), jnp.float32)
    for b in range(n_batches):
        d = xf_ref[b, :] - bc_mu
        if b >= _FULL_BATCHES:
            d = jnp.where(tail_mask, d, jnp.float32(0.0))
        acc2 = acc2 + d * d
    dev2 = jnp.sum(acc2)
    mx = jnp.max(mx_acc)
    mn = jnp.min(mn_acc)
    trend = xf_ref[_FULL_BATCHES, :][_TAIL - 1] - xf_ref[0, :][0]

    out = jnp.full((16,), jnp.float32(0.0))
    out = jnp.where(lane_iota == 0, jnp.full((16,), mean), out)
    out = jnp.where(lane_iota == 1, jnp.full((16,), dev2), out)
    out = jnp.where(lane_iota == 2, jnp.full((16,), mx), out)
    out = jnp.where(lane_iota == 3, jnp.full((16,), mn), out)
    out = jnp.where(lane_iota == 4, jnp.full((16,), trend), out)
    return out


def _sc_stats_kernel(q_hbm, out_hbm, buf0, buf1, outv, scr, xf,
                     sem0, sem1):
    wid = lax.axis_index("c") * NS + lax.axis_index("s")
    base = wid * RPW
    lane_iota = lax.iota(jnp.int32, 16)

    pltpu.make_async_copy(q_hbm.at[base], buf0, sem0).start()
    pltpu.make_async_copy(q_hbm.at[base + 1], buf1, sem1).start()

    def body(i2, _):
        r0 = i2 * 2
        pltpu.make_async_copy(q_hbm.at[base + r0], buf0, sem0).wait()
        outv[r0, :] = _row_stats(buf0, scr, xf, lane_iota)

        @pl.when(r0 + 2 < RPW)
        def _():
            pltpu.make_async_copy(q_hbm.at[base + r0 + 2], buf0,
                                  sem0).start()

        r1 = r0 + 1
        pltpu.make_async_copy(q_hbm.at[base + r1], buf1, sem1).wait()
        outv[r1, :] = _row_stats(buf1, scr, xf, lane_iota)

        @pl.when(r1 + 2 < RPW)
        def _():
            pltpu.make_async_copy(q_hbm.at[base + r1 + 2], buf1,
                                  sem1).start()

        return _

    lax.fori_loop(0, RPW // 2, body, None)
    pltpu.sync_copy(outv, out_hbm.at[pl.ds(base, RPW)])


@jax.jit
def _sc_stats(q2):
    mesh = plsc.VectorSubcoreMesh(core_axis_name="c", subcore_axis_name="s")
    kern = functools.partial(
        pl.kernel,
        out_type=jax.ShapeDtypeStruct((B, 16), jnp.float32),
        mesh=mesh,
        scratch_types=[
            pltpu.VMEM((ROW_W,), jnp.float32),
            pltpu.VMEM((ROW_W,), jnp.float32),
            pltpu.VMEM((RPW, 16), jnp.float32),
            pltpu.VMEM((16, 16), jnp.float32),
            pltpu.VMEM((_FULL_BATCHES + 1, 16), jnp.float32),
            pltpu.SemaphoreType.DMA,
            pltpu.SemaphoreType.DMA,
        ],
        compiler_params=pltpu.CompilerParams(needs_layout_passes=False),
    )(_sc_stats_kernel)
    return kern(q2)


def _fused_kernel(st_ref, w_ref, gbb_ref, mk_ref, mv_ref,
                  ts_ref, rv_ref, bm_ref, mkn_ref):
    i = pl.program_id(0)

    @pl.when(i == 0)
    def _():
        mk = mk_ref[...]
        mkn_ref[...] = (mk / jnp.maximum(
            jnp.sqrt(jnp.sum(mk * mk, axis=1, keepdims=True)),
            jnp.float32(1e-12))).astype(jnp.bfloat16)

    st = st_ref[...]
    mean = st[:, 0:1]
    std = jnp.sqrt(st[:, 1:2] * jnp.float32(1.0 / (T - 1)))
    stats = jnp.concatenate(
        [mean, std, st[:, 2:3], st[:, 3:4], st[:, 4:5]], axis=1)

    h = lax.dot_general(stats.astype(jnp.bfloat16),
                        w_ref[...].astype(jnp.bfloat16),
                        (((1,), (0,)), ((), ())),
                        preferred_element_type=jnp.float32)
    h = h + gbb_ref[0:1, :]
    mu = jnp.mean(h, axis=1, keepdims=True)
    var = jnp.mean((h - mu) ** 2, axis=1, keepdims=True)
    h = (h - mu) * lax.rsqrt(var + jnp.float32(1e-5))
    h = h * gbb_ref[1:2, :] + gbb_ref[2:3, :]

    qn = h / jnp.maximum(jnp.sqrt(jnp.sum(h * h, axis=1, keepdims=True)),
                         jnp.float32(1e-12))

    sim = lax.dot_general(qn.astype(jnp.bfloat16), mkn_ref[...],
                          (((1,), (1,)), ((), ())),
                          preferred_element_type=jnp.float32)

    vals = mv_ref[0:1, :]
    iota = lax.broadcasted_iota(jnp.int32, (BT, MEM), 1)
    work = sim
    top_s = []
    top_v = []
    for _ in range(TOP_K):
        m = jnp.max(work, axis=1, keepdims=True)
        idx = jnp.min(jnp.where(work == m, iota, MEM), axis=1, keepdims=True)
        sel = iota == idx
        v = jnp.sum(jnp.where(sel, vals, jnp.float32(0.0)), axis=1,
                    keepdims=True)
        top_s.append(m)
        top_v.append(v)
        work = jnp.where(sel, _NEG_INF, work)

    ts_ref[...] = jnp.concatenate(top_s, axis=1)
    rv_ref[...] = jnp.concatenate(top_v, axis=1)
    bm_ref[0, 0, 0] = jnp.max(top_s[0])


@jax.jit
def _retrieve(stats16, W, gbb, memory_keys, mv_row):
    grid = B // BT
    ts, rv, bm = pl.pallas_call(
        _fused_kernel,
        grid=(grid,),
        in_specs=[
            pl.BlockSpec((BT, 16), lambda i: (i, 0)),
            pl.BlockSpec((5, D_REPR), lambda i: (0, 0)),
            pl.BlockSpec((3, D_REPR), lambda i: (0, 0)),
            pl.BlockSpec((MEM, D_REPR), lambda i: (0, 0)),
            pl.BlockSpec((1, MEM), lambda i: (0, 0)),
        ],
        out_specs=[
            pl.BlockSpec((BT, TOP_K), lambda i: (i, 0)),
            pl.BlockSpec((BT, TOP_K), lambda i: (i, 0)),
            pl.BlockSpec((1, 1, 1), lambda i: (i, 0, 0),
                         memory_space=pltpu.SMEM),
        ],
        out_shape=[
            jax.ShapeDtypeStruct((B, TOP_K), jnp.float32),
            jax.ShapeDtypeStruct((B, TOP_K), jnp.float32),
            jax.ShapeDtypeStruct((B // BT, 1, 1), jnp.float32),
        ],
        scratch_shapes=[pltpu.VMEM((MEM, D_REPR), jnp.bfloat16)],
    )(stats16, W, gbb, memory_keys, mv_row)
    return ts, rv, bm


def kernel(query, W, b, gamma, beta, memory_keys, memory_values):
    q2 = query.reshape(B, ROW_W)
    gbb = jnp.stack([b, gamma, beta], axis=0)
    mv_row = memory_values.reshape(1, MEM)
    stats16 = _sc_stats(q2)
    ts, rv, bm = _retrieve(stats16, W, gbb, memory_keys, mv_row)
    retrieved_values = rv.reshape(B, TOP_K, 1)
    is_valid = jnp.max(bm) > jnp.float32(THRESH)
    return (retrieved_values, ts, is_valid)


# speedup vs baseline: 1.0586x; 1.0586x over previous
import functools

import jax
import jax.numpy as jnp
from jax import lax
from jax.experimental import pallas as pl
from jax.experimental.pallas import tpu as pltpu
from jax.experimental.pallas import tpu_sc as plsc

D_REPR = 64
TOP_K = 3
THRESH = 0.95
MEM = 5000
T = 200
B = 4096
BT = 256
ROW_W = 12800
S = 2048

NC = 2
NS = 16
NW = NC * NS
RPW = (B - S) // NW

_NEG_INF = float("-inf")

_FULL_BATCHES = T // 16
_TAIL = T - _FULL_BATCHES * 16


def _row_stats(row_ref, scr_ref, xf_ref, lane_iota):
    s_acc = jnp.zeros((16,), jnp.float32)
    mx_acc = jnp.full((16,), _NEG_INF, jnp.float32)
    mn_acc = jnp.full((16,), float("inf"), jnp.float32)
    inv_d = jnp.float32(1.0 / 64.0)
    tail_mask = lane_iota < _TAIL

    n_batches = _FULL_BATCHES + 1
    for b in range(n_batches):
        nk = 16 if b < _FULL_BATCHES else _TAIL
        for k in range(nk):
            base = (b * 16 + k) * 64
            v = (row_ref[pl.ds(base, 16)] + row_ref[pl.ds(base + 16, 16)]
                 + row_ref[pl.ds(base + 32, 16)]
                 + row_ref[pl.ds(base + 48, 16)])
            scr_ref[k, :] = v
        csum = jnp.zeros((16,), jnp.float32)
        colj = jnp.zeros((16,), jnp.int32)
        for j in range(16):
            col = plsc.load_gather(
                scr_ref, [lane_iota, colj + jnp.int32(j)])
            csum = csum + col
        g = csum * inv_d
        if b < _FULL_BATCHES:
            s_acc = s_acc + g
            mx_acc = jnp.maximum(mx_acc, g)
            mn_acc = jnp.minimum(mn_acc, g)
            xf_ref[b, :] = g
        else:
            s_acc = s_acc + jnp.where(tail_mask, g, jnp.float32(0.0))
            mx_acc = jnp.maximum(mx_acc, jnp.where(tail_mask, g, _NEG_INF))
            mn_acc = jnp.minimum(mn_acc,
                                 jnp.where(tail_mask, g, float("inf")))
            xf_ref[b, :] = g

    mean = jnp.sum(s_acc) * jnp.float32(1.0 / T)
    bc_mu = jnp.full((16,), mean, jnp.float32)
    acc2 = jnp.zeros((16,), jnp.float32)
    for b in range(n_batches):
        d = xf_ref[b, :] - bc_mu
        if b >= _FULL_BATCHES:
            d = jnp.where(tail_mask, d, jnp.float32(0.0))
        acc2 = acc2 + d * d
    dev2 = jnp.sum(acc2)
    mx = jnp.max(mx_acc)
    mn = jnp.min(mn_acc)
    trend = xf_ref[_FULL_BATCHES, :][_TAIL - 1] - xf_ref[0, :][0]

    out = jnp.full((16,), jnp.float32(0.0))
    out = jnp.where(lane_iota == 0, jnp.full((16,), mean), out)
    out = jnp.where(lane_iota == 1, jnp.full((16,), dev2), out)
    out = jnp.where(lane_iota == 2, jnp.full((16,), mx), out)
    out = jnp.where(lane_iota == 3, jnp.full((16,), mn), out)
    out = jnp.where(lane_iota == 4, jnp.full((16,), trend), out)
    return out


def _sc_stats_kernel(q_hbm, out_hbm, buf0, buf1, outv, scr, xf,
                     sem0, sem1):
    wid = lax.axis_index("c") * NS + lax.axis_index("s")
    base = S + wid * RPW
    lane_iota = lax.iota(jnp.int32, 16)

    pltpu.make_async_copy(q_hbm.at[base], buf0, sem0).start()
    pltpu.make_async_copy(q_hbm.at[base + 1], buf1, sem1).start()

    def body(i2, _):
        r0 = i2 * 2
        pltpu.make_async_copy(q_hbm.at[base + r0], buf0, sem0).wait()
        outv[r0, :] = _row_stats(buf0, scr, xf, lane_iota)

        @pl.when(r0 + 2 < RPW)
        def _():
            pltpu.make_async_copy(q_hbm.at[base + r0 + 2], buf0,
                                  sem0).start()

        r1 = r0 + 1
        pltpu.make_async_copy(q_hbm.at[base + r1], buf1, sem1).wait()
        outv[r1, :] = _row_stats(buf1, scr, xf, lane_iota)

        @pl.when(r1 + 2 < RPW)
        def _():
            pltpu.make_async_copy(q_hbm.at[base + r1 + 2], buf1,
                                  sem1).start()

        return _

    lax.fori_loop(0, RPW // 2, body, None)
    pltpu.sync_copy(outv, out_hbm.at[pl.ds(wid * RPW, RPW)])


def _sc_stats(q2):
    mesh = plsc.VectorSubcoreMesh(core_axis_name="c", subcore_axis_name="s")
    kern = functools.partial(
        pl.kernel,
        out_type=jax.ShapeDtypeStruct((B - S, 16), jnp.float32),
        mesh=mesh,
        scratch_types=[
            pltpu.VMEM((ROW_W,), jnp.float32),
            pltpu.VMEM((ROW_W,), jnp.float32),
            pltpu.VMEM((RPW, 16), jnp.float32),
            pltpu.VMEM((16, 16), jnp.float32),
            pltpu.VMEM((_FULL_BATCHES + 1, 16), jnp.float32),
            pltpu.SemaphoreType.DMA,
            pltpu.SemaphoreType.DMA,
        ],
        compiler_params=pltpu.CompilerParams(needs_layout_passes=False),
    )(_sc_stats_kernel)
    return kern(q2)


def _norm_mem(mk_ref, mkn_ref):
    mk = mk_ref[...]
    mkn_ref[...] = (mk / jnp.maximum(
        jnp.sqrt(jnp.sum(mk * mk, axis=1, keepdims=True)),
        jnp.float32(1e-12))).astype(jnp.bfloat16)


def _head_from_stats(stats, w_ref, gbb_ref, mv_ref, mkn_ref,
                     ts_ref, rv_ref, bm_ref):
    h = lax.dot_general(stats.astype(jnp.bfloat16),
                        w_ref[...].astype(jnp.bfloat16),
                        (((1,), (0,)), ((), ())),
                        preferred_element_type=jnp.float32)
    h = h + gbb_ref[0:1, :]
    mu = jnp.mean(h, axis=1, keepdims=True)
    var = jnp.mean((h - mu) ** 2, axis=1, keepdims=True)
    h = (h - mu) * lax.rsqrt(var + jnp.float32(1e-5))
    h = h * gbb_ref[1:2, :] + gbb_ref[2:3, :]

    qn = h / jnp.maximum(jnp.sqrt(jnp.sum(h * h, axis=1, keepdims=True)),
                         jnp.float32(1e-12))

    sim = lax.dot_general(qn.astype(jnp.bfloat16), mkn_ref[...],
                          (((1,), (1,)), ((), ())),
                          preferred_element_type=jnp.float32)

    vals = mv_ref[0:1, :]
    iota = lax.broadcasted_iota(jnp.int32, (BT, MEM), 1)
    work = sim
    top_s = []
    top_v = []
    for _ in range(TOP_K):
        m = jnp.max(work, axis=1, keepdims=True)
        idx = jnp.min(jnp.where(work == m, iota, MEM), axis=1, keepdims=True)
        sel = iota == idx
        v = jnp.sum(jnp.where(sel, vals, jnp.float32(0.0)), axis=1,
                    keepdims=True)
        top_s.append(m)
        top_v.append(v)
        work = jnp.where(sel, _NEG_INF, work)

    ts_ref[...] = jnp.concatenate(top_s, axis=1)
    rv_ref[...] = jnp.concatenate(top_v, axis=1)
    bm_ref[0, 0, 0] = jnp.max(top_s[0])


def _fused_full_kernel(q_ref, w_ref, gbb_ref, mk_ref, mv_ref,
                       ts_ref, rv_ref, bm_ref, mkn_ref):
    i = pl.program_id(0)

    @pl.when(i == 0)
    def _():
        _norm_mem(mk_ref, mkn_ref)

    xt = jnp.swapaxes(q_ref[...], 1, 2)
    inv_d = jnp.float32(1.0 / D_REPR)
    ae = jnp.sum(xt[:, :64, :], axis=1) * inv_d
    ao = jnp.sum(xt[:, 64:, :], axis=1) * inv_d

    mean = (jnp.sum(ae, axis=1, keepdims=True)
            + jnp.sum(ao, axis=1, keepdims=True)) * jnp.float32(1.0 / T)
    dev2 = (jnp.sum((ae - mean) ** 2, axis=1, keepdims=True)
            + jnp.sum((ao - mean) ** 2, axis=1, keepdims=True))
    std = jnp.sqrt(dev2 * jnp.float32(1.0 / (T - 1)))
    mx = jnp.maximum(jnp.max(ae, axis=1, keepdims=True),
                     jnp.max(ao, axis=1, keepdims=True))
    mn = jnp.minimum(jnp.min(ae, axis=1, keepdims=True),
                     jnp.min(ao, axis=1, keepdims=True))
    trend = ao[:, T // 2 - 1:] - ae[:, :1]
    stats = jnp.concatenate([mean, std, mx, mn, trend], axis=1)

    _head_from_stats(stats, w_ref, gbb_ref, mv_ref, mkn_ref,
                     ts_ref, rv_ref, bm_ref)


def _fused_slim_kernel(st_ref, w_ref, gbb_ref, mk_ref, mv_ref,
                       ts_ref, rv_ref, bm_ref, mkn_ref):
    i = pl.program_id(0)

    @pl.when(i == 0)
    def _():
        _norm_mem(mk_ref, mkn_ref)

    st = st_ref[...]
    mean = st[:, 0:1]
    std = jnp.sqrt(st[:, 1:2] * jnp.float32(1.0 / (T - 1)))
    stats = jnp.concatenate(
        [mean, std, st[:, 2:3], st[:, 3:4], st[:, 4:5]], axis=1)

    _head_from_stats(stats, w_ref, gbb_ref, mv_ref, mkn_ref,
                     ts_ref, rv_ref, bm_ref)


def _tc_full(q3, W, gbb, memory_keys, mv_row):
    grid = S // BT
    return pl.pallas_call(
        _fused_full_kernel,
        grid=(grid,),
        in_specs=[
            pl.BlockSpec((BT, T // 2, 128), lambda i: (i, 0, 0)),
            pl.BlockSpec((5, D_REPR), lambda i: (0, 0)),
            pl.BlockSpec((3, D_REPR), lambda i: (0, 0)),
            pl.BlockSpec((MEM, D_REPR), lambda i: (0, 0)),
            pl.BlockSpec((1, MEM), lambda i: (0, 0)),
        ],
        out_specs=[
            pl.BlockSpec((BT, TOP_K), lambda i: (i, 0)),
            pl.BlockSpec((BT, TOP_K), lambda i: (i, 0)),
            pl.BlockSpec((1, 1, 1), lambda i: (i, 0, 0),
                         memory_space=pltpu.SMEM),
        ],
        out_shape=[
            jax.ShapeDtypeStruct((S, TOP_K), jnp.float32),
            jax.ShapeDtypeStruct((S, TOP_K), jnp.float32),
            jax.ShapeDtypeStruct((grid, 1, 1), jnp.float32),
        ],
        scratch_shapes=[pltpu.VMEM((MEM, D_REPR), jnp.bfloat16)],
    )(q3, W, gbb, memory_keys, mv_row)


def _tc_slim(stats16, W, gbb, memory_keys, mv_row):
    grid = (B - S) // BT
    return pl.pallas_call(
        _fused_slim_kernel,
        grid=(grid,),
        in_specs=[
            pl.BlockSpec((BT, 16), lambda i: (i, 0)),
            pl.BlockSpec((5, D_REPR), lambda i: (0, 0)),
            pl.BlockSpec((3, D_REPR), lambda i: (0, 0)),
            pl.BlockSpec((MEM, D_REPR), lambda i: (0, 0)),
            pl.BlockSpec((1, MEM), lambda i: (0, 0)),
        ],
        out_specs=[
            pl.BlockSpec((BT, TOP_K), lambda i: (i, 0)),
            pl.BlockSpec((BT, TOP_K), lambda i: (i, 0)),
            pl.BlockSpec((1, 1, 1), lambda i: (i, 0, 0),
                         memory_space=pltpu.SMEM),
        ],
        out_shape=[
            jax.ShapeDtypeStruct((B - S, TOP_K), jnp.float32),
            jax.ShapeDtypeStruct((B - S, TOP_K), jnp.float32),
            jax.ShapeDtypeStruct((grid, 1, 1), jnp.float32),
        ],
        scratch_shapes=[pltpu.VMEM((MEM, D_REPR), jnp.bfloat16)],
    )(stats16, W, gbb, memory_keys, mv_row)


def kernel(query, W, b, gamma, beta, memory_keys, memory_values):
    q2 = query.reshape(B, ROW_W)
    q3 = query.reshape(B, T // 2, 2 * 64)
    gbb = jnp.stack([b, gamma, beta], axis=0)
    mv_row = memory_values.reshape(1, MEM)

    stats16 = _sc_stats(q2)
    tsA, rvA, bmA = _tc_full(q3, W, gbb, memory_keys, mv_row)
    tsB, rvB, bmB = _tc_slim(stats16, W, gbb, memory_keys, mv_row)

    ts = jnp.concatenate([tsA, tsB], axis=0)
    rv = jnp.concatenate([rvA, rvB], axis=0)
    retrieved_values = rv.reshape(B, TOP_K, 1)
    is_valid = jnp.maximum(jnp.max(bmA), jnp.max(bmB)) > jnp.float32(THRESH)
    return (retrieved_values, ts, is_valid)
